# baseline (speedup 1.0000x reference)
import functools

import jax
import jax.numpy as jnp
from jax import lax
from jax.experimental import pallas as pl
from jax.experimental.pallas import tpu as pltpu, tpu_sc as plsc

_BINS = 10
_NC, _NS, _L = 2, 16, 16
_NW = _NC * _NS
_NSAMP = 4194304
_TOTAL = _NSAMP * 2
_CHUNK = 32768
_NCH = 4
_PER_W = _NCH * _CHUNK
_PAIRS = _CHUNK // (2 * _L)
_ROWS = 32768
_SC_ROWS = (_NW * _PER_W) // 256
_TC_R = 64
_TC_G = (_ROWS - _SC_ROWS) // _TC_R

_P4 = (0.10668473260369177, -0.3935358023019218, 0.9797534129748494,
       0.0009250321113056853)
_NLOG2E = -1.4426950408889634


def _ghm_body(x_hbm, t_hbm, out_hbm, xb0, xb1, tb0, tb1, cnt, sm,
              semx0, semx1, semt0, semt1):
    c = lax.axis_index("c")
    s = lax.axis_index("s")
    wid = s * _NC + c
    base = wid * _PER_W
    sbase = wid * (_PER_W // 2)

    xbuf = (xb0, xb1)
    tbuf = (tb0, tb1)
    semx = (semx0, semx1)
    semt = (semt0, semt1)

    zero16 = jnp.zeros((_L,), jnp.float32)
    for r in range(_BINS):
        cnt[r] = zero16
        sm[r] = zero16

    iota = lax.iota(jnp.int32, _L)
    ones = jnp.ones((_L,), jnp.float32)

    def start(g):
        b = g % 2
        dx = pltpu.async_copy(
            x_hbm.at[pl.ds(base + g * _CHUNK, _CHUNK)], xbuf[b], semx[b])
        dt = pltpu.async_copy(
            t_hbm.at[pl.ds(sbase + g * (_CHUNK // 2), _CHUNK // 2)],
            tbuf[b], semt[b])
        return dx, dt

    def accum(xv, yv):
        ax = jnp.abs(xv)
        e = jnp.exp(-ax)
        pos = (xv >= 0.0) != yv
        q = jnp.float32(_BINS - 0.0001) / (1.0 + e)
        g9 = jnp.where(pos, q, jnp.float32(_BINS - 0.0001) - q)
        bin_ = g9.astype(jnp.int32)
        p = jnp.full((_L,), _P4[0], jnp.float32)
        for coef in _P4[1:]:
            p = p * e + jnp.float32(coef)
        bce = jnp.where(pos, ax, 0.0) + p
        plsc.addupdate_scatter(cnt, [bin_, iota], ones)
        plsc.addupdate_scatter(sm, [bin_, iota], bce)

    pending = {0: start(0)}
    for g in range(_NCH):
        b = g % 2
        if g + 1 < _NCH:
            pending[g + 1] = start(g + 1)
        dx, dt = pending.pop(g)
        dx.wait()
        dt.wait()

        @plsc.parallel_loop(0, _PAIRS, unroll=4)
        def inner(i, xr=xbuf[b], tr=tbuf[b]):
            blk = lax.shift_right_logical(i, 3)
            sub = lax.bitwise_and(i, 7)
            off0 = blk * 256 + sub * _L
            x0 = xr[pl.ds(off0, _L)]
            x1 = xr[pl.ds(off0 + 128, _L)]
            tg = tr[pl.ds(i * _L, _L)]
            y0 = tg == 0
            accum(x0, y0)
            accum(x1, jnp.logical_not(y0))

    pltpu.sync_copy(cnt, out_hbm.at[wid, 0])
    pltpu.sync_copy(sm, out_hbm.at[wid, 1])


@functools.partial(
    pl.kernel,
    out_type=jax.ShapeDtypeStruct((_NW, 2, _BINS, _L), jnp.float32),
    mesh=plsc.VectorSubcoreMesh(
        core_axis_name="c", subcore_axis_name="s",
        num_cores=_NC, num_subcores=_NS),
    compiler_params=pltpu.CompilerParams(needs_layout_passes=False),
    scratch_types=[
        pltpu.VMEM((_CHUNK,), jnp.float32),
        pltpu.VMEM((_CHUNK,), jnp.float32),
        pltpu.VMEM((_CHUNK // 2,), jnp.int32),
        pltpu.VMEM((_CHUNK // 2,), jnp.int32),
        pltpu.VMEM((_BINS, _L), jnp.float32),
        pltpu.VMEM((_BINS, _L), jnp.float32),
        pltpu.SemaphoreType.DMA,
        pltpu.SemaphoreType.DMA,
        pltpu.SemaphoreType.DMA,
        pltpu.SemaphoreType.DMA,
    ],
)
def _ghm_pass(x_hbm, t_hbm, out_hbm, *rest):
    _ghm_body(x_hbm, t_hbm, out_hbm, *rest)


def _tc_body(x_ref, t_ref, o_ref):
    c = jnp.float32(_BINS - 0.0001)
    xv = x_ref[...]
    tb = t_ref[...]
    t2 = jnp.broadcast_to(tb[:, None, :], (_TC_R, 2, 128))
    t2 = t2.reshape(2 * _TC_R, 128)
    odd = (lax.broadcasted_iota(jnp.int32, (2 * _TC_R, 128), 0) & 1) == 1
    yv = (t2 == 0) != odd
    ax = jnp.abs(xv)
    e = jnp.exp(-ax)
    pos = (xv >= 0.0) != yv
    d = 1.0 + e
    q = c / d
    g9 = jnp.where(pos, q, c - q)
    bce = jnp.where(pos, ax, 0.0) + jnp.log(d)
    cnts = [jnp.float32(2 * _TC_R * 128)]
    sms = [jnp.sum(bce)]
    for k in range(1, _BINS):
        m = (g9 >= jnp.float32(k)).astype(jnp.float32)
        cnts.append(jnp.sum(m))
        sms.append(jnp.sum(m * bce))
    o_ref[0, 0, :] = jnp.stack(cnts)
    o_ref[0, 1, :] = jnp.stack(sms)


def _ghm_tc(x2d, t2d):
    return pl.pallas_call(
        _tc_body,
        grid=(_TC_G,),
        in_specs=[
            pl.BlockSpec((2 * _TC_R, 128), lambda i: (_SC_ROWS // _TC_R + i, 0)),
            pl.BlockSpec((_TC_R, 128), lambda i: (_SC_ROWS // _TC_R + i, 0)),
        ],
        out_specs=pl.BlockSpec((1, 2, _BINS), lambda i: (i, 0, 0)),
        out_shape=jax.ShapeDtypeStruct((_TC_G, 2, _BINS), jnp.float32),
    )(x2d, t2d)


def kernel(x, target):
    x_lin = x.reshape(32768, 128, 2).transpose(0, 2, 1).reshape(-1)
    tgt = target.astype(jnp.int32)
    parts = _ghm_pass(x_lin, tgt)
    tc = _ghm_tc(x_lin.reshape(2 * _ROWS, 128), tgt.reshape(_ROWS, 128))
    cnt = parts[:, 0].sum(axis=(0, 2))
    sums = parts[:, 1].sum(axis=(0, 2))
    cum = tc.sum(axis=0)
    zero = jnp.zeros((1,), jnp.float32)
    cnt = cnt + cum[0] - jnp.concatenate([cum[0, 1:], zero])
    sums = sums + cum[1] - jnp.concatenate([cum[1, 1:], zero])
    nonempty = jnp.sum(cnt > 0).astype(jnp.float32)
    gd = jnp.clip(cnt * nonempty, 0.0001, None)
    return jnp.sum(sums / gd)


# speedup vs baseline: 1.9425x; 1.9425x over previous
import functools

import jax
import jax.numpy as jnp
from jax import lax
from jax.experimental import pallas as pl
from jax.experimental.pallas import tpu as pltpu, tpu_sc as plsc

_BINS = 10
_NC, _NS, _L = 2, 16, 16
_NW = _NC * _NS
_NSAMP = 4194304
_TOTAL = _NSAMP * 2
_CHUNK = 32768
_NCH = 4
_PER_W = _NCH * _CHUNK
_PAIRS = _CHUNK // (2 * _L)
_ROWS = 32768
_SC_ROWS = (_NW * _PER_W) // 256
_TC_R = 512
_TC_G = (_ROWS - _SC_ROWS) // _TC_R

_P4 = (0.10668473260369177, -0.3935358023019218, 0.9797534129748494,
       0.0009250321113056853)
_NLOG2E = -1.4426950408889634


def _ghm_body(x_hbm, t_hbm, out_hbm, xb0, xb1, tb0, tb1, cnt, sm,
              semx0, semx1, semt0, semt1):
    c = lax.axis_index("c")
    s = lax.axis_index("s")
    wid = s * _NC + c
    base = wid * _PER_W
    sbase = wid * (_PER_W // 2)

    xbuf = (xb0, xb1)
    tbuf = (tb0, tb1)
    semx = (semx0, semx1)
    semt = (semt0, semt1)

    zero16 = jnp.zeros((_L,), jnp.float32)
    for r in range(_BINS):
        cnt[r] = zero16
        sm[r] = zero16

    iota = lax.iota(jnp.int32, _L)
    ones = jnp.ones((_L,), jnp.float32)

    def start(g):
        b = g % 2
        dx = pltpu.async_copy(
            x_hbm.at[pl.ds(base + g * _CHUNK, _CHUNK)], xbuf[b], semx[b])
        dt = pltpu.async_copy(
            t_hbm.at[pl.ds(sbase + g * (_CHUNK // 2), _CHUNK // 2)],
            tbuf[b], semt[b])
        return dx, dt

    def accum(xv, yv):
        ax = jnp.abs(xv)
        e = jnp.exp(-ax)
        pos = (xv >= 0.0) != yv
        q = jnp.float32(_BINS - 0.0001) / (1.0 + e)
        g9 = jnp.where(pos, q, jnp.float32(_BINS - 0.0001) - q)
        bin_ = g9.astype(jnp.int32)
        p = jnp.full((_L,), _P4[0], jnp.float32)
        for coef in _P4[1:]:
            p = p * e + jnp.float32(coef)
        bce = jnp.where(pos, ax, 0.0) + p
        plsc.addupdate_scatter(cnt, [bin_, iota], ones)
        plsc.addupdate_scatter(sm, [bin_, iota], bce)

    pending = {0: start(0)}
    for g in range(_NCH):
        b = g % 2
        if g + 1 < _NCH:
            pending[g + 1] = start(g + 1)
        dx, dt = pending.pop(g)
        dx.wait()
        dt.wait()

        @plsc.parallel_loop(0, _PAIRS, unroll=4)
        def inner(i, xr=xbuf[b], tr=tbuf[b]):
            blk = lax.shift_right_logical(i, 3)
            sub = lax.bitwise_and(i, 7)
            off0 = blk * 256 + sub * _L
            x0 = xr[pl.ds(off0, _L)]
            x1 = xr[pl.ds(off0 + 128, _L)]
            tg = tr[pl.ds(i * _L, _L)]
            y0 = tg == 0
            accum(x0, y0)
            accum(x1, jnp.logical_not(y0))

    pltpu.sync_copy(cnt, out_hbm.at[wid, 0])
    pltpu.sync_copy(sm, out_hbm.at[wid, 1])


@functools.partial(
    pl.kernel,
    out_type=jax.ShapeDtypeStruct((_NW, 2, _BINS, _L), jnp.float32),
    mesh=plsc.VectorSubcoreMesh(
        core_axis_name="c", subcore_axis_name="s",
        num_cores=_NC, num_subcores=_NS),
    compiler_params=pltpu.CompilerParams(needs_layout_passes=False),
    scratch_types=[
        pltpu.VMEM((_CHUNK,), jnp.float32),
        pltpu.VMEM((_CHUNK,), jnp.float32),
        pltpu.VMEM((_CHUNK // 2,), jnp.int32),
        pltpu.VMEM((_CHUNK // 2,), jnp.int32),
        pltpu.VMEM((_BINS, _L), jnp.float32),
        pltpu.VMEM((_BINS, _L), jnp.float32),
        pltpu.SemaphoreType.DMA,
        pltpu.SemaphoreType.DMA,
        pltpu.SemaphoreType.DMA,
        pltpu.SemaphoreType.DMA,
    ],
)
def _ghm_pass(x_hbm, t_hbm, out_hbm, *rest):
    _ghm_body(x_hbm, t_hbm, out_hbm, *rest)


def _tc_body(x_ref, t_ref, o_ref):
    c = jnp.float32(_BINS - 0.0001)
    xv = x_ref[...]
    tb = t_ref[...]
    t2 = jnp.broadcast_to(tb[:, None, :], (_TC_R, 2, 128))
    t2 = t2.reshape(2 * _TC_R, 128)
    odd = (lax.broadcasted_iota(jnp.int32, (2 * _TC_R, 128), 0) & 1) == 1
    yv = (t2 == 0) != odd
    ax = jnp.abs(xv)
    e = jnp.exp(-ax)
    pos = (xv >= 0.0) != yv
    d = 1.0 + e
    q = c / d
    g9 = jnp.where(pos, q, c - q)
    bce = jnp.where(pos, ax, 0.0) + jnp.log(d)
    cnts = [jnp.float32(2 * _TC_R * 128)]
    sms = [jnp.sum(bce)]
    for k in range(1, _BINS):
        m = (g9 >= jnp.float32(k)).astype(jnp.float32)
        cnts.append(jnp.sum(m))
        sms.append(jnp.sum(m * bce))
    o_ref[0, 0, :] = jnp.stack(cnts)
    o_ref[0, 1, :] = jnp.stack(sms)


def _ghm_tc(x2d, t2d):
    return pl.pallas_call(
        _tc_body,
        grid=(_TC_G,),
        in_specs=[
            pl.BlockSpec((2 * _TC_R, 128), lambda i: (_SC_ROWS // _TC_R + i, 0)),
            pl.BlockSpec((_TC_R, 128), lambda i: (_SC_ROWS // _TC_R + i, 0)),
        ],
        out_specs=pl.BlockSpec((1, 2, _BINS), lambda i: (i, 0, 0)),
        out_shape=jax.ShapeDtypeStruct((_TC_G, 2, _BINS), jnp.float32),
    )(x2d, t2d)


def kernel(x, target):
    x_lin = x.reshape(32768, 128, 2).transpose(0, 2, 1).reshape(-1)
    tgt = target.astype(jnp.int32)
    parts = _ghm_pass(x_lin, tgt)
    tc = _ghm_tc(x_lin.reshape(2 * _ROWS, 128), tgt.reshape(_ROWS, 128))
    cnt = parts[:, 0].sum(axis=(0, 2))
    sums = parts[:, 1].sum(axis=(0, 2))
    cum = tc.sum(axis=0)
    zero = jnp.zeros((1,), jnp.float32)
    cnt = cnt + cum[0] - jnp.concatenate([cum[0, 1:], zero])
    sums = sums + cum[1] - jnp.concatenate([cum[1, 1:], zero])
    nonempty = jnp.sum(cnt > 0).astype(jnp.float32)
    gd = jnp.clip(cnt * nonempty, 0.0001, None)
    return jnp.sum(sums / gd)


# speedup vs baseline: 2.2439x; 1.1551x over previous
import functools

import jax
import jax.numpy as jnp
from jax import lax
from jax.experimental import pallas as pl
from jax.experimental.pallas import tpu as pltpu, tpu_sc as plsc

_BINS = 10
_NC, _NS, _L = 2, 16, 16
_NW = _NC * _NS
_NSAMP = 4194304
_TOTAL = _NSAMP * 2
_CHUNK = 32768
_NCH = 5
_PER_W = _NCH * _CHUNK
_PAIRS = _CHUNK // (2 * _L)
_ROWS = 32768
_SC_ROWS = (_NW * _PER_W) // 256
_TC_R = 512
_TC_G = (_ROWS - _SC_ROWS) // _TC_R

_P4 = (0.10668473260369177, -0.3935358023019218, 0.9797534129748494,
       0.0009250321113056853)
_NLOG2E = -1.4426950408889634


def _ghm_body(x_hbm, t_hbm, out_hbm, xb0, xb1, tb0, tb1, cnt, sm,
              semx0, semx1, semt0, semt1):
    c = lax.axis_index("c")
    s = lax.axis_index("s")
    wid = s * _NC + c
    base = wid * _PER_W
    sbase = wid * (_PER_W // 2)

    xbuf = (xb0, xb1)
    tbuf = (tb0, tb1)
    semx = (semx0, semx1)
    semt = (semt0, semt1)

    zero16 = jnp.zeros((_L,), jnp.float32)
    for r in range(_BINS):
        cnt[r] = zero16
        sm[r] = zero16

    iota = lax.iota(jnp.int32, _L)
    ones = jnp.ones((_L,), jnp.float32)

    def start(g):
        b = g % 2
        dx = pltpu.async_copy(
            x_hbm.at[pl.ds(base + g * _CHUNK, _CHUNK)], xbuf[b], semx[b])
        dt = pltpu.async_copy(
            t_hbm.at[pl.ds(sbase + g * (_CHUNK // 2), _CHUNK // 2)],
            tbuf[b], semt[b])
        return dx, dt

    def accum(xv, yv):
        ax = jnp.abs(xv)
        e = jnp.exp(-ax)
        pos = (xv >= 0.0) != yv
        q = jnp.float32(_BINS - 0.0001) / (1.0 + e)
        g9 = jnp.where(pos, q, jnp.float32(_BINS - 0.0001) - q)
        bin_ = g9.astype(jnp.int32)
        p = jnp.full((_L,), _P4[0], jnp.float32)
        for coef in _P4[1:]:
            p = p * e + jnp.float32(coef)
        bce = jnp.where(pos, ax, 0.0) + p
        plsc.addupdate_scatter(cnt, [bin_, iota], ones)
        plsc.addupdate_scatter(sm, [bin_, iota], bce)

    pending = {0: start(0)}
    for g in range(_NCH):
        b = g % 2
        if g + 1 < _NCH:
            pending[g + 1] = start(g + 1)
        dx, dt = pending.pop(g)
        dx.wait()
        dt.wait()

        @plsc.parallel_loop(0, _PAIRS, unroll=4)
        def inner(i, xr=xbuf[b], tr=tbuf[b]):
            blk = lax.shift_right_logical(i, 3)
            sub = lax.bitwise_and(i, 7)
            off0 = blk * 256 + sub * _L
            x0 = xr[pl.ds(off0, _L)]
            x1 = xr[pl.ds(off0 + 128, _L)]
            tg = tr[pl.ds(i * _L, _L)]
            y0 = tg == 0
            accum(x0, y0)
            accum(x1, jnp.logical_not(y0))

    pltpu.sync_copy(cnt, out_hbm.at[wid, 0])
    pltpu.sync_copy(sm, out_hbm.at[wid, 1])


@functools.partial(
    pl.kernel,
    out_type=jax.ShapeDtypeStruct((_NW, 2, _BINS, _L), jnp.float32),
    mesh=plsc.VectorSubcoreMesh(
        core_axis_name="c", subcore_axis_name="s",
        num_cores=_NC, num_subcores=_NS),
    compiler_params=pltpu.CompilerParams(needs_layout_passes=False),
    scratch_types=[
        pltpu.VMEM((_CHUNK,), jnp.float32),
        pltpu.VMEM((_CHUNK,), jnp.float32),
        pltpu.VMEM((_CHUNK // 2,), jnp.int32),
        pltpu.VMEM((_CHUNK // 2,), jnp.int32),
        pltpu.VMEM((_BINS, _L), jnp.float32),
        pltpu.VMEM((_BINS, _L), jnp.float32),
        pltpu.SemaphoreType.DMA,
        pltpu.SemaphoreType.DMA,
        pltpu.SemaphoreType.DMA,
        pltpu.SemaphoreType.DMA,
    ],
)
def _ghm_pass(x_hbm, t_hbm, out_hbm, *rest):
    _ghm_body(x_hbm, t_hbm, out_hbm, *rest)


def _tc_body(x_ref, t_ref, o_ref):
    c = jnp.float32(_BINS - 0.0001)
    xv = x_ref[...]
    tb = t_ref[...]
    t2 = jnp.broadcast_to(tb[:, None, :], (_TC_R, 2, 128))
    t2 = t2.reshape(2 * _TC_R, 128)
    odd = (lax.broadcasted_iota(jnp.int32, (2 * _TC_R, 128), 0) & 1) == 1
    yv = (t2 == 0) != odd
    ax = jnp.abs(xv)
    e = jnp.exp(-ax)
    pos = (xv >= 0.0) != yv
    d = 1.0 + e
    q = c / d
    g9 = jnp.where(pos, q, c - q)
    p = jnp.full(e.shape, _P4[0], jnp.float32)
    for coef in _P4[1:]:
        p = p * e + jnp.float32(coef)
    bce = jnp.where(pos, ax, 0.0) + p
    cnts = [jnp.float32(2 * _TC_R * 128)]
    sms = [jnp.sum(bce)]
    for k in range(1, _BINS):
        m = (g9 >= jnp.float32(k)).astype(jnp.float32)
        cnts.append(jnp.sum(m))
        sms.append(jnp.sum(m * bce))
    o_ref[0, 0, :] = jnp.stack(cnts)
    o_ref[0, 1, :] = jnp.stack(sms)


def _ghm_tc(x2d, t2d):
    return pl.pallas_call(
        _tc_body,
        grid=(_TC_G,),
        in_specs=[
            pl.BlockSpec((2 * _TC_R, 128), lambda i: (_SC_ROWS // _TC_R + i, 0)),
            pl.BlockSpec((_TC_R, 128), lambda i: (_SC_ROWS // _TC_R + i, 0)),
        ],
        out_specs=pl.BlockSpec((1, 2, _BINS), lambda i: (i, 0, 0)),
        out_shape=jax.ShapeDtypeStruct((_TC_G, 2, _BINS), jnp.float32),
    )(x2d, t2d)


def kernel(x, target):
    x_lin = x.reshape(32768, 128, 2).transpose(0, 2, 1).reshape(-1)
    tgt = target.astype(jnp.int32)
    parts = _ghm_pass(x_lin, tgt)
    tc = _ghm_tc(x_lin.reshape(2 * _ROWS, 128), tgt.reshape(_ROWS, 128))
    cnt = parts[:, 0].sum(axis=(0, 2))
    sums = parts[:, 1].sum(axis=(0, 2))
    cum = tc.sum(axis=0)
    zero = jnp.zeros((1,), jnp.float32)
    cnt = cnt + cum[0] - jnp.concatenate([cum[0, 1:], zero])
    sums = sums + cum[1] - jnp.concatenate([cum[1, 1:], zero])
    nonempty = jnp.sum(cnt > 0).astype(jnp.float32)
    gd = jnp.clip(cnt * nonempty, 0.0001, None)
    return jnp.sum(sums / gd)
